# baseline (speedup 1.0000x reference)
import jax
import jax.numpy as jnp
from jax import lax
from jax.experimental import pallas as pl
from jax.experimental.pallas import tpu as pltpu

N_ROIS = 200
CIN = 256
P = 14
W16 = 16
R = 240
B = 8
RB = B * R
G = 16


def _head_kernel(x_ref, wc_ref, bc_ref, wt_ref, bt_ref, w5_ref, b5_ref, o_ref,
                 s_ref):
    rows = lax.broadcasted_iota(jnp.int32, (RB, 1), 0) % R
    live = (rows < P * W16) & (rows % W16 < P)
    zero = jnp.zeros((), jnp.bfloat16)

    s_ref[:, 0:G + 1, :] = jnp.zeros((2, G + 1, 3 * CIN), jnp.bfloat16)
    s_ref[:, G + RB - 1:, :] = jnp.zeros((2, G + 1, 3 * CIN), jnp.bfloat16)

    def store_panes(buf, c0, xb):
        n = xb.shape[0]
        s_ref[buf, G + c0:G + c0 + n, CIN:2 * CIN] = xb
        s_ref[buf, G + c0 + 1:G + c0 + 1 + n, 0:CIN] = xb
        s_ref[buf, G + c0 - 1:G + c0 - 1 + n, 2 * CIN:3 * CIN] = xb

    xt = jnp.transpose(x_ref[...], (0, 2, 1)).astype(jnp.bfloat16)
    gap = jnp.zeros((B, W16, CIN), jnp.bfloat16)
    chunks = [jnp.pad(xt[:, P * h:P * h + P, :], ((0, 0), (0, W16 - P), (0, 0)))
              for h in range(P)]
    x0 = jnp.concatenate(chunks + [gap], axis=1).reshape(RB, CIN)
    for c in range(4):
        store_panes(0, c * (RB // 4), x0[c * (RB // 4):(c + 1) * (RB // 4)])

    for li in range(4):
        src, dst = li % 2, (li + 1) % 2
        acc = jnp.broadcast_to(bc_ref[li][None, :], (RB, CIN))
        for t in range(9):
            dy, dx = t // 3 - 1, t % 3 - 1
            a = s_ref[src, G + W16 * dy:G + W16 * dy + RB,
                      CIN * (dx + 1):CIN * (dx + 2)]
            acc = acc + jnp.dot(a, wc_ref[li, t],
                                preferred_element_type=jnp.float32)
        xb = jnp.where(live, jax.nn.relu(acc).astype(jnp.bfloat16), zero)
        store_panes(dst, 0, xb)

    NB = B // 2
    for c in range(NB):
        xc = s_ref[0, G + c * 2 * R:G + (c + 1) * 2 * R, CIN:2 * CIN]
        up = jnp.dot(xc, wt_ref[...], preferred_element_type=jnp.float32)
        up = jax.nn.relu(up + bt_ref[...]).astype(jnp.bfloat16)
        y = jnp.dot(up, w5_ref[...], preferred_element_type=jnp.float32)
        y = jax.nn.sigmoid(y + b5_ref[...])
        o_ref[2 * c:2 * c + 2, :, :] = y.reshape(2, R, 12)


def kernel(features, w1, b1, w2, b2, w3, b3, w4, b4, wt, bt, w5, b5):
    fx = features.reshape(N_ROIS, CIN, P * P)

    wc = jnp.stack([jnp.transpose(w, (2, 3, 1, 0)).reshape(9, CIN, CIN)
                    for w in (w1, w2, w3, w4)]).astype(jnp.bfloat16)
    bc = jnp.stack([b1, b2, b3, b4])
    wtm = jnp.transpose(wt, (2, 3, 0, 1)).reshape(4, CIN, CIN)
    wtm = jnp.concatenate([wtm[t] for t in range(4)], axis=1).astype(jnp.bfloat16)
    bt4 = jnp.tile(bt, 4)[None, :]
    w5m = jnp.transpose(w5[:, :, 0, 0])
    w5b = jnp.zeros((4 * CIN, 12), jnp.float32)
    for t in range(4):
        w5b = w5b.at[t * CIN:(t + 1) * CIN, t * 3:(t + 1) * 3].set(w5m)
    w5b = w5b.astype(jnp.bfloat16)
    b5b = jnp.tile(b5, 4)[None, :]

    out = pl.pallas_call(
        _head_kernel,
        grid=(N_ROIS // B,),
        in_specs=[
            pl.BlockSpec((B, CIN, P * P), lambda i: (i, 0, 0)),
            pl.BlockSpec((4, 9, CIN, CIN), lambda i: (0, 0, 0, 0)),
            pl.BlockSpec((4, CIN), lambda i: (0, 0)),
            pl.BlockSpec((CIN, 4 * CIN), lambda i: (0, 0)),
            pl.BlockSpec((1, 4 * CIN), lambda i: (0, 0)),
            pl.BlockSpec((4 * CIN, 12), lambda i: (0, 0)),
            pl.BlockSpec((1, 12), lambda i: (0, 0)),
        ],
        out_specs=pl.BlockSpec((B, R, 12), lambda i: (i, 0, 0)),
        out_shape=jax.ShapeDtypeStruct((N_ROIS, R, 12), jnp.float32),
        scratch_shapes=[pltpu.VMEM((2, G + RB + G, 3 * CIN), jnp.bfloat16)],
        compiler_params=pltpu.CompilerParams(
            dimension_semantics=("parallel",)),
    )(fx, wc, bc, wtm, bt4, w5b, b5b)

    o = out[:, :P * W16, :].reshape(N_ROIS, P, W16, 12)[:, :, :P, :]
    o = o.reshape(N_ROIS, P, P, 2, 2, 3)
    return o.transpose(0, 5, 1, 3, 2, 4).reshape(N_ROIS, 3, 2 * P, 2 * P)


# speedup vs baseline: 1.5446x; 1.5446x over previous
import jax
import jax.numpy as jnp
from jax import lax
from jax.experimental import pallas as pl
from jax.experimental.pallas import tpu as pltpu

N_ROIS = 200
CIN = 256
P = 14
W16 = 16
R = 240
B = 8
RB = B * R
G = 16


def _head_kernel(x_ref, wc_ref, bc_ref, wt_ref, bt_ref, w5_ref, b5_ref, o_ref,
                 s_ref):
    rows = lax.broadcasted_iota(jnp.int32, (RB, 1), 0) % R
    live = (rows < P * W16) & (rows % W16 < P)
    zero = jnp.zeros((), jnp.bfloat16)

    s_ref[:, 0:G + 1, :] = jnp.zeros((2, G + 1, 3 * CIN), jnp.bfloat16)
    s_ref[:, G + RB - 1:, :] = jnp.zeros((2, G + 1, 3 * CIN), jnp.bfloat16)

    def store_panes(buf, c0, xb):
        n = xb.shape[0]
        s_ref[buf, G + c0:G + c0 + n, CIN:2 * CIN] = xb
        s_ref[buf, G + c0 + 1:G + c0 + 1 + n, 0:CIN] = xb
        s_ref[buf, G + c0 - 1:G + c0 - 1 + n, 2 * CIN:3 * CIN] = xb

    xt = jnp.transpose(x_ref[...], (0, 2, 1)).astype(jnp.bfloat16)
    gap = jnp.zeros((B, W16, CIN), jnp.bfloat16)
    chunks = [jnp.pad(xt[:, P * h:P * h + P, :], ((0, 0), (0, W16 - P), (0, 0)))
              for h in range(P)]
    x0 = jnp.concatenate(chunks + [gap], axis=1).reshape(RB, CIN)
    for c in range(4):
        store_panes(0, c * (RB // 4), x0[c * (RB // 4):(c + 1) * (RB // 4)])

    CH = 15
    CR = RB // CH
    for li in range(4):
        src, dst = li % 2, (li + 1) % 2
        for c in range(CH):
            c0 = c * CR
            x9 = jnp.concatenate(
                [s_ref[src, G + c0 - W16:G + c0 - W16 + CR, :],
                 s_ref[src, G + c0:G + c0 + CR, :],
                 s_ref[src, G + c0 + W16:G + c0 + W16 + CR, :]],
                axis=1)
            acc = jnp.dot(x9, wc_ref[li], preferred_element_type=jnp.float32)
            acc = acc + bc_ref[li][None, :]
            xb = jnp.where(live[c0:c0 + CR], jax.nn.relu(acc).astype(jnp.bfloat16),
                           zero)
            store_panes(dst, c0, xb)

    NB = B // 2
    for c in range(NB):
        xc = s_ref[0, G + c * 2 * R:G + (c + 1) * 2 * R, CIN:2 * CIN]
        up = jnp.dot(xc, wt_ref[...], preferred_element_type=jnp.float32)
        up = jax.nn.relu(up + bt_ref[...]).astype(jnp.bfloat16)
        y = jnp.dot(up, w5_ref[...], preferred_element_type=jnp.float32)
        y = jax.nn.sigmoid(y + b5_ref[...])
        o_ref[2 * c:2 * c + 2, :, :] = y.reshape(2, R, 12)


def kernel(features, w1, b1, w2, b2, w3, b3, w4, b4, wt, bt, w5, b5):
    fx = features.reshape(N_ROIS, CIN, P * P)

    wc = jnp.stack([jnp.transpose(w, (2, 3, 1, 0)).reshape(9 * CIN, CIN)
                    for w in (w1, w2, w3, w4)]).astype(jnp.bfloat16)
    bc = jnp.stack([b1, b2, b3, b4])
    wtm = jnp.transpose(wt, (2, 3, 0, 1)).reshape(4, CIN, CIN)
    wtm = jnp.concatenate([wtm[t] for t in range(4)], axis=1).astype(jnp.bfloat16)
    bt4 = jnp.tile(bt, 4)[None, :]
    w5m = jnp.transpose(w5[:, :, 0, 0])
    w5b = jnp.zeros((4 * CIN, 12), jnp.float32)
    for t in range(4):
        w5b = w5b.at[t * CIN:(t + 1) * CIN, t * 3:(t + 1) * 3].set(w5m)
    w5b = w5b.astype(jnp.bfloat16)
    b5b = jnp.tile(b5, 4)[None, :]

    out = pl.pallas_call(
        _head_kernel,
        grid=(N_ROIS // B,),
        in_specs=[
            pl.BlockSpec((B, CIN, P * P), lambda i: (i, 0, 0)),
            pl.BlockSpec((4, 9 * CIN, CIN), lambda i: (0, 0, 0)),
            pl.BlockSpec((4, CIN), lambda i: (0, 0)),
            pl.BlockSpec((CIN, 4 * CIN), lambda i: (0, 0)),
            pl.BlockSpec((1, 4 * CIN), lambda i: (0, 0)),
            pl.BlockSpec((4 * CIN, 12), lambda i: (0, 0)),
            pl.BlockSpec((1, 12), lambda i: (0, 0)),
        ],
        out_specs=pl.BlockSpec((B, R, 12), lambda i: (i, 0, 0)),
        out_shape=jax.ShapeDtypeStruct((N_ROIS, R, 12), jnp.float32),
        scratch_shapes=[pltpu.VMEM((2, G + RB + G, 3 * CIN), jnp.bfloat16)],
        compiler_params=pltpu.CompilerParams(
            dimension_semantics=("parallel",)),
    )(fx, wc, bc, wtm, bt4, w5b, b5b)

    o = out[:, :P * W16, :].reshape(N_ROIS, P, W16, 12)[:, :, :P, :]
    o = o.reshape(N_ROIS, P, P, 2, 2, 3)
    return o.transpose(0, 5, 1, 3, 2, 4).reshape(N_ROIS, 3, 2 * P, 2 * P)
